# baseline (speedup 1.0000x reference)
import functools

import jax
import jax.numpy as jnp
from jax import lax
from jax.experimental import pallas as pl
from jax.experimental.pallas import tpu as pltpu
from jax.experimental.pallas import tpu_sc as plsc

VOCAB = 100000
EMBED_DIM = 300
NUM_LABELS = 2
BATCH = 4096
SEQ = 200

NUM_CORES = 2
NUM_SUBCORES = 16
LANES = 16

B_PER_TILE = BATCH // NUM_SUBCORES
GROUPS = B_PER_TILE // LANES
L_CHUNK = 40
N_CHUNKS = SEQ // L_CHUNK

V_BLOCK = 10000


def _proj_body(w_ref, e_ref, out_ref):
    out_ref[...] = lax.dot_general(
        e_ref[...], w_ref[...],
        (((1,), (1,)), ((), ())),
        preferred_element_type=jnp.float32,
    )


def _project_table(embed_weight, cls_w):
    grid = (VOCAB // V_BLOCK,)
    return pl.pallas_call(
        _proj_body,
        grid=grid,
        in_specs=[
            pl.BlockSpec((NUM_LABELS, EMBED_DIM), lambda i: (0, 0)),
            pl.BlockSpec((V_BLOCK, EMBED_DIM), lambda i: (i, 0)),
        ],
        out_specs=pl.BlockSpec((V_BLOCK, NUM_LABELS), lambda i: (i, 0)),
        out_shape=jax.ShapeDtypeStruct((VOCAB, NUM_LABELS), jnp.float32),
    )(cls_w, embed_weight)


R_CHUNK = 64
N_RCHUNKS = B_PER_TILE // R_CHUNK
G_PER_CHUNK = R_CHUNK // LANES


def _sc_pool_body(p_hbm, ids_hbm, out_hbm, col_v, idx_v, out_v, sem):
    c = lax.axis_index("c")
    s = lax.axis_index("s")

    pltpu.sync_copy(p_hbm.at[c], col_v)

    lane_iota = lax.iota(jnp.int32, LANES)
    for t in range(N_RCHUNKS):
        base = s * B_PER_TILE + t * R_CHUNK
        pltpu.sync_copy(ids_hbm.at[pl.ds(base, R_CHUNK)], idx_v)

        rows = [g * LANES + lane_iota for g in range(G_PER_CHUNK)]

        def body(l, accs):
            lvec = jnp.full((LANES,), l, jnp.int32)
            return tuple(
                acc + plsc.load_gather(
                    col_v, [plsc.load_gather(idx_v, [rows[g], lvec])])
                for g, acc in enumerate(accs)
            )

        zero = jnp.zeros((LANES,), jnp.float32)
        accs = lax.fori_loop(0, SEQ, body, (zero,) * G_PER_CHUNK)
        for g in range(G_PER_CHUNK):
            out_v[pl.ds(t * R_CHUNK + g * LANES, LANES)] = accs[g]

    pltpu.sync_copy(out_v, out_hbm.at[c, pl.ds(s * B_PER_TILE, B_PER_TILE)])


def _sc_pool(p_t, ids):
    mesh = plsc.VectorSubcoreMesh(core_axis_name="c", subcore_axis_name="s")
    fn = functools.partial(
        pl.kernel,
        mesh=mesh,
        out_type=jax.ShapeDtypeStruct((NUM_LABELS, BATCH), jnp.float32),
        scratch_types=[
            pltpu.VMEM((VOCAB,), jnp.float32),
            pltpu.VMEM((R_CHUNK, SEQ), jnp.int32),
            pltpu.VMEM((B_PER_TILE,), jnp.float32),
            pltpu.SemaphoreType.DMA,
        ],
        compiler_params=pltpu.CompilerParams(needs_layout_passes=False),
    )(_sc_pool_body)
    return fn(p_t, ids)


def kernel(input_ids, embed_weight, cls_w, cls_b):
    p_t = _project_table(embed_weight, cls_w).T
    out_t = _sc_pool(p_t, input_ids)
    return out_t.T + cls_b[None, :]


# speedup vs baseline: 1.0794x; 1.0794x over previous
import functools

import jax
import jax.numpy as jnp
from jax import lax
from jax.experimental import pallas as pl
from jax.experimental.pallas import tpu as pltpu
from jax.experimental.pallas import tpu_sc as plsc

VOCAB = 100000
EMBED_DIM = 300
NUM_LABELS = 2
BATCH = 4096
SEQ = 200

NUM_CORES = 2
NUM_SUBCORES = 16
LANES = 16

B_PER_TILE = BATCH // NUM_SUBCORES
GROUPS = B_PER_TILE // LANES
L_CHUNK = 40
N_CHUNKS = SEQ // L_CHUNK

V_BLOCK = 10000


def _proj_body(w_ref, e_ref, out_ref):
    out_ref[...] = lax.dot_general(
        e_ref[...], w_ref[...],
        (((1,), (1,)), ((), ())),
        preferred_element_type=jnp.float32,
    )


def _project_table(embed_weight, cls_w):
    grid = (VOCAB // V_BLOCK,)
    return pl.pallas_call(
        _proj_body,
        grid=grid,
        in_specs=[
            pl.BlockSpec((NUM_LABELS, EMBED_DIM), lambda i: (0, 0)),
            pl.BlockSpec((V_BLOCK, EMBED_DIM), lambda i: (i, 0)),
        ],
        out_specs=pl.BlockSpec((V_BLOCK, NUM_LABELS), lambda i: (i, 0)),
        out_shape=jax.ShapeDtypeStruct((VOCAB, NUM_LABELS), jnp.float32),
    )(cls_w, embed_weight)


R_CHUNK = 64
N_RCHUNKS = B_PER_TILE // R_CHUNK
G_PER_CHUNK = R_CHUNK // LANES


def _sc_pool_body(p_hbm, ids_hbm, out_hbm, col_v, idx_a, idx_b, out_v,
                  sem_col, sem_a, sem_b):
    c = lax.axis_index("c")
    s = lax.axis_index("s")
    base = s * B_PER_TILE

    col_cp = pltpu.async_copy(p_hbm.at[c], col_v, sem_col)
    bufs = [(idx_a, sem_a), (idx_b, sem_b)]
    cps = [None] * N_RCHUNKS
    for t in range(2):
        buf, sem = bufs[t % 2]
        cps[t] = pltpu.async_copy(
            ids_hbm.at[pl.ds((base + t * R_CHUNK) * SEQ, R_CHUNK * SEQ)],
            buf, sem)
    col_cp.wait()

    lane_iota = lax.iota(jnp.int32, LANES)
    rowoff = [(g * LANES + lane_iota) * SEQ for g in range(G_PER_CHUNK)]
    zero = jnp.zeros((LANES,), jnp.float32)

    for t in range(N_RCHUNKS):
        buf, sem = bufs[t % 2]
        cps[t].wait()

        def body(l, accs, idx_v=buf):
            lvec = jnp.full((LANES,), l, jnp.int32)
            return tuple(
                acc + plsc.load_gather(
                    col_v, [plsc.load_gather(idx_v, [rowoff[g] + lvec])])
                for g, acc in enumerate(accs)
            )

        accs = plsc.parallel_loop(
            0, SEQ, unroll=4, carry=(zero,) * G_PER_CHUNK)(body)
        for g in range(G_PER_CHUNK):
            out_v[pl.ds(t * R_CHUNK + g * LANES, LANES)] = accs[g]

        if t + 2 < N_RCHUNKS:
            cps[t + 2] = pltpu.async_copy(
                ids_hbm.at[pl.ds((base + (t + 2) * R_CHUNK) * SEQ,
                                 R_CHUNK * SEQ)],
                buf, sem)

    pltpu.sync_copy(out_v, out_hbm.at[c, pl.ds(s * B_PER_TILE, B_PER_TILE)])


def _sc_pool(p_t, ids):
    mesh = plsc.VectorSubcoreMesh(core_axis_name="c", subcore_axis_name="s")
    fn = functools.partial(
        pl.kernel,
        mesh=mesh,
        out_type=jax.ShapeDtypeStruct((NUM_LABELS, BATCH), jnp.float32),
        scratch_types=[
            pltpu.VMEM((VOCAB,), jnp.float32),
            pltpu.VMEM((R_CHUNK * SEQ,), jnp.int32),
            pltpu.VMEM((R_CHUNK * SEQ,), jnp.int32),
            pltpu.VMEM((B_PER_TILE,), jnp.float32),
            pltpu.SemaphoreType.DMA,
            pltpu.SemaphoreType.DMA,
            pltpu.SemaphoreType.DMA,
        ],
        compiler_params=pltpu.CompilerParams(needs_layout_passes=False),
    )(_sc_pool_body)
    return fn(p_t, ids.reshape(BATCH * SEQ))


def kernel(input_ids, embed_weight, cls_w, cls_b):
    p_t = _project_table(embed_weight, cls_w).T
    out_t = _sc_pool(p_t, input_ids)
    return out_t.T + cls_b[None, :]


# speedup vs baseline: 1.1559x; 1.0708x over previous
import functools

import jax
import jax.numpy as jnp
from jax import lax
from jax.experimental import pallas as pl
from jax.experimental.pallas import tpu as pltpu
from jax.experimental.pallas import tpu_sc as plsc

VOCAB = 100000
EMBED_DIM = 300
NUM_LABELS = 2
BATCH = 4096
SEQ = 200

NUM_CORES = 2
NUM_SUBCORES = 16
LANES = 16

B_PER_TILE = BATCH // NUM_SUBCORES
GROUPS = B_PER_TILE // LANES
L_CHUNK = 40
N_CHUNKS = SEQ // L_CHUNK

V_BLOCK = 10000


P_CHUNK = 128
N_FULL = VOCAB // P_CHUNK
K_HI_TILES = 13
K_LO = 24
PADV = (N_FULL + 1) * P_CHUNK
NKC = EMBED_DIM // LANES
TAIL_OFF = EMBED_DIM - LANES
PAIRS = (K_LO + 1 + 1) // 2


def _sc_proj_body(e_hbm, w_hbm, p_hbm, ea, eb, w_v, out_v,
                  sem_a, sem_b):
    c = lax.axis_index("c")
    s = lax.axis_index("s")
    wid = s * NUM_CORES + c
    k_n = jnp.where(wid < K_HI_TILES, K_LO + 1, K_LO)
    base = (K_LO + 1) * wid - jnp.maximum(wid - K_HI_TILES, 0)

    pltpu.sync_copy(w_hbm, w_v)

    def chunk_src(n):
        return e_hbm.at[pl.ds(P_CHUNK * (base + n), P_CHUNK)]

    pltpu.async_copy(chunk_src(0), ea, sem_a)
    pltpu.async_copy(chunk_src(1), eb, sem_b)

    lane_iota = lax.iota(jnp.int32, LANES)
    offs = [LANES * k for k in range(NKC)] + [TAIL_OFF]
    wvecs = []
    for j in range(NUM_LABELS):
        row = [w_v[j, pl.ds(o, LANES)] for o in offs]
        row[NKC] = jnp.where(lane_iota >= 4, row[NKC], 0.0)
        wvecs.append(row)

    def group_body(n, e_buf, g):
        vecs = [jnp.zeros((LANES,), jnp.float32)
                for _ in range(NUM_LABELS)]
        for j in range(LANES):
            r = g * LANES + j
            eks = [e_buf[r, pl.ds(o, LANES)] for o in offs]
            for lab in range(NUM_LABELS):
                prods = [e * w for e, w in zip(eks, wvecs[lab])]
                while len(prods) > 1:
                    tailp = [prods[-1]] if len(prods) % 2 else []
                    prods = [a + b for a, b in
                             zip(prods[0::2], prods[1::2])] + tailp
                sc = jnp.sum(prods[0])
                vecs[lab] = jnp.where(lane_iota == j, sc, vecs[lab])
        off = n * P_CHUNK + g * LANES
        for lab in range(NUM_LABELS):
            out_v[lab, pl.ds(off, LANES)] = vecs[lab]

    def do_chunk(n, e_buf):
        plsc.parallel_loop(0, P_CHUNK // LANES)(
            lambda g: group_body(n, e_buf, g))

    def do_chunk_static(n, e_buf, ngroups):
        for g in range(ngroups):
            group_body(n, e_buf, g)

    def pair(t, carry):
        na = 2 * t

        @pl.when(na < k_n)
        def _():
            pltpu.make_async_copy(chunk_src(na), ea, sem_a).wait()
            do_chunk(na, ea)

            @pl.when(na + 2 < k_n)
            def _():
                pltpu.async_copy(chunk_src(na + 2), ea, sem_a)

        @pl.when(na + 1 < k_n)
        def _():
            pltpu.make_async_copy(chunk_src(na + 1), eb, sem_b).wait()
            do_chunk(na + 1, eb)

            @pl.when(na + 3 < k_n)
            def _():
                pltpu.async_copy(chunk_src(na + 3), eb, sem_b)

        return carry

    lax.fori_loop(0, PAIRS, pair, 0)

    n_tail = VOCAB - N_FULL * P_CHUNK

    @pl.when(wid == 31)
    def _():
        pltpu.sync_copy(e_hbm.at[pl.ds(N_FULL * P_CHUNK, n_tail)],
                        ea.at[pl.ds(0, n_tail)])
        do_chunk_static(K_LO, ea, n_tail // LANES)

    rows_lo = P_CHUNK * K_LO

    @pl.when(jnp.logical_or(wid < K_HI_TILES, wid == 31))
    def _():
        pltpu.sync_copy(
            out_v.at[:, pl.ds(0, rows_lo + P_CHUNK)],
            p_hbm.at[:, pl.ds(P_CHUNK * base, rows_lo + P_CHUNK)])

    @pl.when(jnp.logical_and(wid >= K_HI_TILES, wid < 31))
    def _():
        pltpu.sync_copy(
            out_v.at[:, pl.ds(0, rows_lo)],
            p_hbm.at[:, pl.ds(P_CHUNK * base, rows_lo)])


def _project_table(embed_weight, cls_w):
    mesh = plsc.VectorSubcoreMesh(core_axis_name="c", subcore_axis_name="s")
    fn = functools.partial(
        pl.kernel,
        mesh=mesh,
        out_type=jax.ShapeDtypeStruct((NUM_LABELS, PADV), jnp.float32),
        scratch_types=[
            pltpu.VMEM((P_CHUNK, EMBED_DIM), jnp.float32),
            pltpu.VMEM((P_CHUNK, EMBED_DIM), jnp.float32),
            pltpu.VMEM((NUM_LABELS, EMBED_DIM), jnp.float32),
            pltpu.VMEM((NUM_LABELS, P_CHUNK * (K_LO + 1)), jnp.float32),
            pltpu.SemaphoreType.DMA,
            pltpu.SemaphoreType.DMA,
        ],
        compiler_params=pltpu.CompilerParams(needs_layout_passes=False),
    )(_sc_proj_body)
    return fn(embed_weight, cls_w)


R_CHUNK = 64
N_RCHUNKS = B_PER_TILE // R_CHUNK
G_PER_CHUNK = R_CHUNK // LANES


def _sc_pool_body(p_hbm, ids_hbm, out_hbm, col_v, idx_a, idx_b, out_v,
                  sem_col, sem_a, sem_b):
    c = lax.axis_index("c")
    s = lax.axis_index("s")
    base = s * B_PER_TILE

    col_cp = pltpu.async_copy(p_hbm.at[c], col_v, sem_col)
    bufs = [(idx_a, sem_a), (idx_b, sem_b)]
    cps = [None] * N_RCHUNKS
    for t in range(2):
        buf, sem = bufs[t % 2]
        cps[t] = pltpu.async_copy(
            ids_hbm.at[pl.ds((base + t * R_CHUNK) * SEQ, R_CHUNK * SEQ)],
            buf, sem)
    col_cp.wait()

    lane_iota = lax.iota(jnp.int32, LANES)
    rowoff = [(g * LANES + lane_iota) * SEQ for g in range(G_PER_CHUNK)]
    zero = jnp.zeros((LANES,), jnp.float32)

    for t in range(N_RCHUNKS):
        buf, sem = bufs[t % 2]
        cps[t].wait()

        def body(l, accs, idx_v=buf):
            lvec = jnp.full((LANES,), l, jnp.int32)
            return tuple(
                acc + plsc.load_gather(
                    col_v, [plsc.load_gather(idx_v, [rowoff[g] + lvec])])
                for g, acc in enumerate(accs)
            )

        accs = plsc.parallel_loop(
            0, SEQ, unroll=4, carry=(zero,) * G_PER_CHUNK)(body)
        for g in range(G_PER_CHUNK):
            out_v[pl.ds(t * R_CHUNK + g * LANES, LANES)] = accs[g]

        if t + 2 < N_RCHUNKS:
            cps[t + 2] = pltpu.async_copy(
                ids_hbm.at[pl.ds((base + (t + 2) * R_CHUNK) * SEQ,
                                 R_CHUNK * SEQ)],
                buf, sem)

    pltpu.sync_copy(out_v, out_hbm.at[c, pl.ds(s * B_PER_TILE, B_PER_TILE)])


def _sc_pool(p_t, ids):
    mesh = plsc.VectorSubcoreMesh(core_axis_name="c", subcore_axis_name="s")
    fn = functools.partial(
        pl.kernel,
        mesh=mesh,
        out_type=jax.ShapeDtypeStruct((NUM_LABELS, BATCH), jnp.float32),
        scratch_types=[
            pltpu.VMEM((PADV,), jnp.float32),
            pltpu.VMEM((R_CHUNK * SEQ,), jnp.int32),
            pltpu.VMEM((R_CHUNK * SEQ,), jnp.int32),
            pltpu.VMEM((B_PER_TILE,), jnp.float32),
            pltpu.SemaphoreType.DMA,
            pltpu.SemaphoreType.DMA,
            pltpu.SemaphoreType.DMA,
        ],
        compiler_params=pltpu.CompilerParams(needs_layout_passes=False),
    )(_sc_pool_body)
    return fn(p_t, ids.reshape(BATCH * SEQ))


def kernel(input_ids, embed_weight, cls_w, cls_b):
    p_t = _project_table(embed_weight, cls_w)
    out_t = _sc_pool(p_t, input_ids)
    return out_t.T + cls_b[None, :]


# speedup vs baseline: 1.1610x; 1.0044x over previous
import functools

import jax
import jax.numpy as jnp
from jax import lax
from jax.experimental import pallas as pl
from jax.experimental.pallas import tpu as pltpu
from jax.experimental.pallas import tpu_sc as plsc

VOCAB = 100000
EMBED_DIM = 300
NUM_LABELS = 2
BATCH = 4096
SEQ = 200

NUM_CORES = 2
NUM_SUBCORES = 16
LANES = 16

B_PER_TILE = BATCH // NUM_SUBCORES
GROUPS = B_PER_TILE // LANES
L_CHUNK = 40
N_CHUNKS = SEQ // L_CHUNK

V_BLOCK = 10000


P_CHUNK = 128
N_FULL = VOCAB // P_CHUNK
K_HI_TILES = 13
K_LO = 24
PADV = (N_FULL + 1) * P_CHUNK
NKC = EMBED_DIM // LANES
TAIL_OFF = EMBED_DIM - LANES
PAIRS = (K_LO + 1 + 1) // 2


def _sc_proj_body(e_hbm, w_hbm, p_hbm, ea, eb, w_v, out_v,
                  sem_a, sem_b):
    c = lax.axis_index("c")
    s = lax.axis_index("s")
    wid = s * NUM_CORES + c
    k_n = jnp.where(wid < K_HI_TILES, K_LO + 1, K_LO)
    base = (K_LO + 1) * wid - jnp.maximum(wid - K_HI_TILES, 0)

    pltpu.sync_copy(w_hbm, w_v)

    def chunk_src(n):
        return e_hbm.at[pl.ds(P_CHUNK * (base + n), P_CHUNK)]

    pltpu.async_copy(chunk_src(0), ea, sem_a)
    pltpu.async_copy(chunk_src(1), eb, sem_b)

    lane_iota = lax.iota(jnp.int32, LANES)
    offs = [LANES * k for k in range(NKC)] + [TAIL_OFF]
    wvecs = []
    for j in range(NUM_LABELS):
        row = [w_v[j, pl.ds(o, LANES)] for o in offs]
        row[NKC] = jnp.where(lane_iota >= 4, row[NKC], 0.0)
        wvecs.append(row)

    def group_body(n, e_buf, g):
        vecs = [jnp.zeros((LANES,), jnp.float32)
                for _ in range(NUM_LABELS)]
        for j in range(LANES):
            r = g * LANES + j
            eks = [e_buf[r, pl.ds(o, LANES)] for o in offs]
            for lab in range(NUM_LABELS):
                prods = [e * w for e, w in zip(eks, wvecs[lab])]
                while len(prods) > 1:
                    tailp = [prods[-1]] if len(prods) % 2 else []
                    prods = [a + b for a, b in
                             zip(prods[0::2], prods[1::2])] + tailp
                sc = jnp.sum(prods[0])
                vecs[lab] = jnp.where(lane_iota == j, sc, vecs[lab])
        off = n * P_CHUNK + g * LANES
        for lab in range(NUM_LABELS):
            out_v[lab, pl.ds(off, LANES)] = vecs[lab]

    def do_chunk(n, e_buf):
        plsc.parallel_loop(0, P_CHUNK // LANES)(
            lambda g: group_body(n, e_buf, g))

    def do_chunk_static(n, e_buf, ngroups):
        for g in range(ngroups):
            group_body(n, e_buf, g)

    def pair(t, carry):
        na = 2 * t

        @pl.when(na < k_n)
        def _():
            pltpu.make_async_copy(chunk_src(na), ea, sem_a).wait()
            do_chunk(na, ea)

            @pl.when(na + 2 < k_n)
            def _():
                pltpu.async_copy(chunk_src(na + 2), ea, sem_a)

        @pl.when(na + 1 < k_n)
        def _():
            pltpu.make_async_copy(chunk_src(na + 1), eb, sem_b).wait()
            do_chunk(na + 1, eb)

            @pl.when(na + 3 < k_n)
            def _():
                pltpu.async_copy(chunk_src(na + 3), eb, sem_b)

        return carry

    lax.fori_loop(0, PAIRS, pair, 0)

    n_tail = VOCAB - N_FULL * P_CHUNK

    @pl.when(wid == 31)
    def _():
        pltpu.sync_copy(e_hbm.at[pl.ds(N_FULL * P_CHUNK, n_tail)],
                        ea.at[pl.ds(0, n_tail)])
        do_chunk_static(K_LO, ea, n_tail // LANES)

    rows_lo = P_CHUNK * K_LO

    @pl.when(jnp.logical_or(wid < K_HI_TILES, wid == 31))
    def _():
        pltpu.sync_copy(
            out_v.at[:, pl.ds(0, rows_lo + P_CHUNK)],
            p_hbm.at[:, pl.ds(P_CHUNK * base, rows_lo + P_CHUNK)])

    @pl.when(jnp.logical_and(wid >= K_HI_TILES, wid < 31))
    def _():
        pltpu.sync_copy(
            out_v.at[:, pl.ds(0, rows_lo)],
            p_hbm.at[:, pl.ds(P_CHUNK * base, rows_lo)])


def _project_table(embed_weight, cls_w):
    mesh = plsc.VectorSubcoreMesh(core_axis_name="c", subcore_axis_name="s")
    fn = functools.partial(
        pl.kernel,
        mesh=mesh,
        out_type=jax.ShapeDtypeStruct((NUM_LABELS, PADV), jnp.float32),
        scratch_types=[
            pltpu.VMEM((P_CHUNK, EMBED_DIM), jnp.float32),
            pltpu.VMEM((P_CHUNK, EMBED_DIM), jnp.float32),
            pltpu.VMEM((NUM_LABELS, EMBED_DIM), jnp.float32),
            pltpu.VMEM((NUM_LABELS, P_CHUNK * (K_LO + 1)), jnp.float32),
            pltpu.SemaphoreType.DMA,
            pltpu.SemaphoreType.DMA,
        ],
        compiler_params=pltpu.CompilerParams(
            needs_layout_passes=False, use_tc_tiling_on_sc=True),
    )(_sc_proj_body)
    return fn(embed_weight, cls_w)


R_CHUNK = 64
N_RCHUNKS = B_PER_TILE // R_CHUNK
G_PER_CHUNK = R_CHUNK // LANES


def _sc_pool_body(p_hbm, ids_hbm, out_hbm, col_v, idx_a, idx_b, out_v,
                  sem_col, sem_a, sem_b):
    c = lax.axis_index("c")
    s = lax.axis_index("s")
    base = s * B_PER_TILE

    col_cp = pltpu.async_copy(p_hbm.at[c], col_v, sem_col)
    bufs = [(idx_a, sem_a), (idx_b, sem_b)]
    cps = [None] * N_RCHUNKS
    for t in range(2):
        buf, sem = bufs[t % 2]
        cps[t] = pltpu.async_copy(
            ids_hbm.at[pl.ds((base + t * R_CHUNK) * SEQ, R_CHUNK * SEQ)],
            buf, sem)
    col_cp.wait()

    lane_iota = lax.iota(jnp.int32, LANES)
    rowoff = [(g * LANES + lane_iota) * SEQ for g in range(G_PER_CHUNK)]
    zero = jnp.zeros((LANES,), jnp.float32)

    for t in range(N_RCHUNKS):
        buf, sem = bufs[t % 2]
        cps[t].wait()

        def body(l, accs, idx_v=buf):
            lvec = jnp.full((LANES,), l, jnp.int32)
            return tuple(
                acc + plsc.load_gather(
                    col_v, [plsc.load_gather(idx_v, [rowoff[g] + lvec])])
                for g, acc in enumerate(accs)
            )

        accs = plsc.parallel_loop(
            0, SEQ, unroll=4, carry=(zero,) * G_PER_CHUNK)(body)
        for g in range(G_PER_CHUNK):
            out_v[pl.ds(t * R_CHUNK + g * LANES, LANES)] = accs[g]

        if t + 2 < N_RCHUNKS:
            cps[t + 2] = pltpu.async_copy(
                ids_hbm.at[pl.ds((base + (t + 2) * R_CHUNK) * SEQ,
                                 R_CHUNK * SEQ)],
                buf, sem)

    pltpu.sync_copy(out_v, out_hbm.at[c, pl.ds(s * B_PER_TILE, B_PER_TILE)])


def _sc_pool(p_t, ids):
    mesh = plsc.VectorSubcoreMesh(core_axis_name="c", subcore_axis_name="s")
    fn = functools.partial(
        pl.kernel,
        mesh=mesh,
        out_type=jax.ShapeDtypeStruct((NUM_LABELS, BATCH), jnp.float32),
        scratch_types=[
            pltpu.VMEM((PADV,), jnp.float32),
            pltpu.VMEM((R_CHUNK * SEQ,), jnp.int32),
            pltpu.VMEM((R_CHUNK * SEQ,), jnp.int32),
            pltpu.VMEM((B_PER_TILE,), jnp.float32),
            pltpu.SemaphoreType.DMA,
            pltpu.SemaphoreType.DMA,
            pltpu.SemaphoreType.DMA,
        ],
        compiler_params=pltpu.CompilerParams(needs_layout_passes=False),
    )(_sc_pool_body)
    return fn(p_t, ids.reshape(BATCH * SEQ))


def kernel(input_ids, embed_weight, cls_w, cls_b):
    p_t = _project_table(embed_weight, cls_w)
    out_t = _sc_pool(p_t, input_ids)
    return out_t.T + cls_b[None, :]
